# Initial kernel scaffold; baseline (speedup 1.0000x reference)
import jax
import jax.numpy as jnp
from jax.experimental import pallas as pl


def kernel(emb, bboxes, person_mask, W1e, b1e, W2e, b2e, W1n, b1n, W2n, b2n, gamma, beta):
    raise NotImplementedError("write your pallas kernel here")


# speedup vs baseline: 12.5729x; 12.5729x over previous
import functools

import jax
import jax.numpy as jnp
from jax.experimental import pallas as pl

K_NN = 8
RADIUS = 2.5
_BIG = 1000000.0


def _graph_body(cxr, cyr, mr, cxc, cyc, hc, mc, x_ref,
                w1ea, w1eb, wc, b1e, w2e, b2e,
                w1na, w1nb, b1n, w2n, b2n, gam, bet,
                out_ref):
    G, N, D = x_ref.shape
    H = w1ea.shape[1]

    dxT = cxc[...] - cxr[...]
    dyT = cyc[...] - cyr[...]
    distT = jnp.sqrt(dxT * dxT + dyT * dyT + 1e-6)
    hcv = hc[...]
    dxnT = dxT / hcv
    dynT = dyT / hcv
    distnT = distT / hcv

    jj = jax.lax.broadcasted_iota(jnp.int32, (G, N, N), 1)
    ii = jax.lax.broadcasted_iota(jnp.int32, (G, N, N), 2)
    validT = (mr[...] * mc[...] > 0.0) & (jj != ii)
    dknnT = jnp.where(validT, distnT, _BIG)

    rank = jnp.zeros((G, N, N), dtype=jnp.int32)
    for k in range(N):
        rowk = dknnT[:, k:k + 1, :]
        cmp = (rowk < dknnT) | ((rowk == dknnT) & (k < jj))
        rank = rank + cmp.astype(jnp.int32)
    selT = validT & (rank < K_NN) & (distnT < RADIUS)
    wT = selT.astype(jnp.float32)

    x2 = x_ref[...].reshape(G * N, D)
    a = jnp.dot(x2, w1ea[...], preferred_element_type=jnp.float32)
    b = jnp.dot(x2, w1eb[...], preferred_element_type=jnp.float32)
    a3 = a.reshape(G, N, H)
    b3 = b.reshape(G, N, H) + b1e[...]
    wc0 = wc[0:1, :].reshape(1, 1, H)
    wc1 = wc[1:2, :].reshape(1, 1, H)
    wc2 = wc[2:3, :].reshape(1, 1, H)

    s_parts = []
    c_parts = []
    for i in range(N):
        ai = a3[:, i:i + 1, :]
        ei = (dxnT[:, :, i:i + 1] * wc0 + dynT[:, :, i:i + 1] * wc1
              + distnT[:, :, i:i + 1] * wc2)
        h1 = jnp.maximum(ai + b3 + ei, 0.0)
        wi = wT[:, :, i:i + 1]
        s_parts.append(jnp.sum(wi * h1, axis=1, keepdims=True))
        c_parts.append(jnp.sum(wi, axis=1, keepdims=True))
    s = jnp.concatenate(s_parts, axis=1).reshape(G * N, H)
    cnt = jnp.concatenate(c_parts, axis=1).reshape(G * N, 1)

    denom = jnp.maximum(cnt, 1.0)
    hasn = (cnt > 0.0).astype(jnp.float32)
    agg = jnp.dot(s, w2e[...], preferred_element_type=jnp.float32) / denom \
        + b2e[...] * hasn
    n1 = jnp.maximum(
        jnp.dot(x2, w1na[...], preferred_element_type=jnp.float32)
        + jnp.dot(agg, w1nb[...], preferred_element_type=jnp.float32)
        + b1n[...], 0.0)
    delta = (jnp.dot(n1, w2n[...], preferred_element_type=jnp.float32)
             + b2n[...]) * hasn
    y = x2 + delta
    mu = jnp.mean(y, axis=1, keepdims=True)
    yc = y - mu
    var = jnp.mean(yc * yc, axis=1, keepdims=True)
    out = yc / jnp.sqrt(var + 1e-5) * gam[...] + bet[...]
    out = out * mr[...].reshape(G * N, 1)
    out_ref[...] = out.reshape(G, N, D)


@functools.partial(jax.jit, static_argnames=("interpret",))
def kernel(emb, bboxes, person_mask, W1e, b1e, W2e, b2e, W1n, b1n, W2n, b2n,
           gamma, beta, interpret=False):
    B, T, N, D = emb.shape
    BT = B * T
    H = W1e.shape[1]
    G = 8
    x = emb.reshape(BT, N, D)
    boxes = bboxes.reshape(BT, N, 4)
    cx = boxes[:, :, 0]
    cy = boxes[:, :, 1]
    h = jnp.maximum(boxes[:, :, 3], 1e-6)
    m = person_mask.reshape(BT, N).astype(jnp.float32)
    cxr, cyr, mr = cx[:, :, None], cy[:, :, None], m[:, :, None]
    cxc, cyc, hc, mc = cx[:, None, :], cy[:, None, :], h[:, None, :], m[:, None, :]

    row = pl.BlockSpec((G, N, 1), lambda g: (g, 0, 0))
    col = pl.BlockSpec((G, 1, N), lambda g: (g, 0, 0))
    xsp = pl.BlockSpec((G, N, D), lambda g: (g, 0, 0))

    def full(arr):
        return pl.BlockSpec(arr.shape, lambda g: (0,) * arr.ndim)

    w1ea, w1eb, wce = W1e[:D], W1e[D:2 * D], W1e[2 * D:]
    w1na, w1nb = W1n[:D], W1n[D:]
    wts = (w1ea, w1eb, wce, b1e.reshape(1, H), W2e, b2e.reshape(1, D),
           w1na, w1nb, b1n.reshape(1, H), W2n, b2n.reshape(1, D),
           gamma.reshape(1, D), beta.reshape(1, D))

    out = pl.pallas_call(
        _graph_body,
        grid=(BT // G,),
        in_specs=[row, row, row, col, col, col, col, xsp] + [full(w) for w in wts],
        out_specs=xsp,
        out_shape=jax.ShapeDtypeStruct((BT, N, D), jnp.float32),
        interpret=interpret,
    )(cxr, cyr, mr, cxc, cyc, hc, mc, x, *wts)
    return out.reshape(B, T, N, D)


# speedup vs baseline: 14.3312x; 1.1398x over previous
import functools

import jax
import jax.numpy as jnp
from jax.experimental import pallas as pl

K_NN = 8
RADIUS = 2.5
_BIG = 1000000.0


def _graph_body(cxr, cyr, mr, cxc, cyc, hc, mc, x_ref,
                w1ea, w1eb, wc, b1e, w2e, b2e,
                w1na, w1nb, b1n, w2n, b2n, gam, bet,
                out_ref):
    G, N, D = x_ref.shape
    H = w1ea.shape[1]

    dxT = cxc[...] - cxr[...]
    dyT = cyc[...] - cyr[...]
    distT = jnp.sqrt(dxT * dxT + dyT * dyT + 1e-6)
    hcv = hc[...]
    dxnT = dxT / hcv
    dynT = dyT / hcv
    distnT = distT / hcv

    jj = jax.lax.broadcasted_iota(jnp.int32, (G, N, N), 1)
    ii = jax.lax.broadcasted_iota(jnp.int32, (G, N, N), 2)
    validT = (mr[...] * mc[...] > 0.0) & (jj != ii)
    dknnT = jnp.where(validT, distnT, _BIG)

    rank = jnp.zeros((G, N, N), dtype=jnp.int32)
    for k in range(N):
        rowk = dknnT[:, k:k + 1, :]
        cmp = (rowk < dknnT) | ((rowk == dknnT) & (k < jj))
        rank = rank + cmp.astype(jnp.int32)
    selT = validT & (rank < K_NN) & (distnT < RADIUS)
    wT = selT.astype(jnp.float32)

    x2 = x_ref[...].reshape(G * N, D)
    a = jnp.dot(x2, w1ea[...], preferred_element_type=jnp.float32)
    b = jnp.dot(x2, w1eb[...], preferred_element_type=jnp.float32)
    a3 = a.reshape(G, N, H)
    b3 = b.reshape(G, N, H) + b1e[...]
    wc0 = wc[0:1, :].reshape(1, 1, H)
    wc1 = wc[1:2, :].reshape(1, 1, H)
    wc2 = wc[2:3, :].reshape(1, 1, H)

    s_parts = []
    c_parts = []
    for i in range(N):
        ai = a3[:, i:i + 1, :]
        ei = (dxnT[:, :, i:i + 1] * wc0 + dynT[:, :, i:i + 1] * wc1
              + distnT[:, :, i:i + 1] * wc2)
        h1 = jnp.maximum(ai + b3 + ei, 0.0)
        wi = wT[:, :, i:i + 1]
        s_parts.append(jnp.sum(wi * h1, axis=1, keepdims=True))
        c_parts.append(jnp.sum(wi, axis=1, keepdims=True))
    s = jnp.concatenate(s_parts, axis=1).reshape(G * N, H)
    cnt = jnp.concatenate(c_parts, axis=1).reshape(G * N, 1)

    denom = jnp.maximum(cnt, 1.0)
    hasn = (cnt > 0.0).astype(jnp.float32)
    agg = jnp.dot(s, w2e[...], preferred_element_type=jnp.float32) / denom \
        + b2e[...] * hasn
    n1 = jnp.maximum(
        jnp.dot(x2, w1na[...], preferred_element_type=jnp.float32)
        + jnp.dot(agg, w1nb[...], preferred_element_type=jnp.float32)
        + b1n[...], 0.0)
    delta = (jnp.dot(n1, w2n[...], preferred_element_type=jnp.float32)
             + b2n[...]) * hasn
    y = x2 + delta
    mu = jnp.mean(y, axis=1, keepdims=True)
    yc = y - mu
    var = jnp.mean(yc * yc, axis=1, keepdims=True)
    out = yc / jnp.sqrt(var + 1e-5) * gam[...] + bet[...]
    out = out * mr[...].reshape(G * N, 1)
    out_ref[...] = out.reshape(G, N, D)


@functools.partial(jax.jit, static_argnames=("interpret",))
def kernel(emb, bboxes, person_mask, W1e, b1e, W2e, b2e, W1n, b1n, W2n, b2n,
           gamma, beta, interpret=False):
    B, T, N, D = emb.shape
    BT = B * T
    H = W1e.shape[1]
    G = 32
    x = emb.reshape(BT, N, D)
    boxes = bboxes.reshape(BT, N, 4)
    cx = boxes[:, :, 0]
    cy = boxes[:, :, 1]
    h = jnp.maximum(boxes[:, :, 3], 1e-6)
    m = person_mask.reshape(BT, N).astype(jnp.float32)
    cxr, cyr, mr = cx[:, :, None], cy[:, :, None], m[:, :, None]
    cxc, cyc, hc, mc = cx[:, None, :], cy[:, None, :], h[:, None, :], m[:, None, :]

    row = pl.BlockSpec((G, N, 1), lambda g: (g, 0, 0))
    col = pl.BlockSpec((G, 1, N), lambda g: (g, 0, 0))
    xsp = pl.BlockSpec((G, N, D), lambda g: (g, 0, 0))

    def full(arr):
        return pl.BlockSpec(arr.shape, lambda g: (0,) * arr.ndim)

    w1ea, w1eb, wce = W1e[:D], W1e[D:2 * D], W1e[2 * D:]
    w1na, w1nb = W1n[:D], W1n[D:]
    wts = (w1ea, w1eb, wce, b1e.reshape(1, H), W2e, b2e.reshape(1, D),
           w1na, w1nb, b1n.reshape(1, H), W2n, b2n.reshape(1, D),
           gamma.reshape(1, D), beta.reshape(1, D))

    out = pl.pallas_call(
        _graph_body,
        grid=(BT // G,),
        in_specs=[row, row, row, col, col, col, col, xsp] + [full(w) for w in wts],
        out_specs=xsp,
        out_shape=jax.ShapeDtypeStruct((BT, N, D), jnp.float32),
        interpret=interpret,
    )(cxr, cyr, mr, cxc, cyc, hc, mc, x, *wts)
    return out.reshape(B, T, N, D)


# speedup vs baseline: 14.5213x; 1.0133x over previous
import functools

import jax
import jax.numpy as jnp
from jax.experimental import pallas as pl

K_NN = 8
RADIUS = 2.5
_BIG = 1000000.0


def _graph_body(cxr, cyr, mr, cxc, cyc, hc, mc, x_ref,
                w1ea, w1eb, wc, b1e, w2e, b2e,
                w1na, w1nb, b1n, w2n, b2n, gam, bet,
                out_ref):
    G, N, D = x_ref.shape
    H = w1ea.shape[1]

    dxT = cxc[...] - cxr[...]
    dyT = cyc[...] - cyr[...]
    distT = jnp.sqrt(dxT * dxT + dyT * dyT + 1e-6)
    hcv = hc[...]
    dxnT = dxT / hcv
    dynT = dyT / hcv
    distnT = distT / hcv

    jj = jax.lax.broadcasted_iota(jnp.int32, (G, N, N), 1)
    ii = jax.lax.broadcasted_iota(jnp.int32, (G, N, N), 2)
    validT = (mr[...] * mc[...] > 0.0) & (jj != ii)
    dknnT = jnp.where(validT, distnT, _BIG)

    rank = jnp.zeros((G, N, N), dtype=jnp.int32)
    for k in range(N):
        rowk = dknnT[:, k:k + 1, :]
        cmp = (rowk < dknnT) | ((rowk == dknnT) & (k < jj))
        rank = rank + cmp.astype(jnp.int32)
    selT = validT & (rank < K_NN) & (distnT < RADIUS)
    wT = selT.astype(jnp.float32)

    x2 = x_ref[...].reshape(G * N, D)
    a = jnp.dot(x2, w1ea[...], preferred_element_type=jnp.float32)
    b = jnp.dot(x2, w1eb[...], preferred_element_type=jnp.float32)
    a3 = a.reshape(G, N, H)
    b3 = b.reshape(G, N, H) + b1e[...]
    wc0 = wc[0:1, :].reshape(1, 1, H)
    wc1 = wc[1:2, :].reshape(1, 1, H)
    wc2 = wc[2:3, :].reshape(1, 1, H)

    s_parts = []
    c_parts = []
    for i in range(N):
        ai = a3[:, i:i + 1, :]
        ei = (dxnT[:, :, i:i + 1] * wc0 + dynT[:, :, i:i + 1] * wc1
              + distnT[:, :, i:i + 1] * wc2)
        h1 = jnp.maximum(ai + b3 + ei, 0.0)
        wi = wT[:, :, i:i + 1]
        s_parts.append(jnp.sum(wi * h1, axis=1, keepdims=True))
        c_parts.append(jnp.sum(wi, axis=1, keepdims=True))
    s = jnp.concatenate(s_parts, axis=1).reshape(G * N, H)
    cnt = jnp.concatenate(c_parts, axis=1).reshape(G * N, 1)

    denom = jnp.maximum(cnt, 1.0)
    hasn = (cnt > 0.0).astype(jnp.float32)
    agg = jnp.dot(s, w2e[...], preferred_element_type=jnp.float32) / denom \
        + b2e[...] * hasn
    n1 = jnp.maximum(
        jnp.dot(x2, w1na[...], preferred_element_type=jnp.float32)
        + jnp.dot(agg, w1nb[...], preferred_element_type=jnp.float32)
        + b1n[...], 0.0)
    delta = (jnp.dot(n1, w2n[...], preferred_element_type=jnp.float32)
             + b2n[...]) * hasn
    y = x2 + delta
    mu = jnp.mean(y, axis=1, keepdims=True)
    yc = y - mu
    var = jnp.mean(yc * yc, axis=1, keepdims=True)
    out = yc / jnp.sqrt(var + 1e-5) * gam[...] + bet[...]
    out = out * mr[...].reshape(G * N, 1)
    out_ref[...] = out.reshape(G, N, D)


@functools.partial(jax.jit, static_argnames=("interpret",))
def kernel(emb, bboxes, person_mask, W1e, b1e, W2e, b2e, W1n, b1n, W2n, b2n,
           gamma, beta, interpret=False):
    B, T, N, D = emb.shape
    BT = B * T
    H = W1e.shape[1]
    G = 64
    x = emb.reshape(BT, N, D)
    boxes = bboxes.reshape(BT, N, 4)
    cx = boxes[:, :, 0]
    cy = boxes[:, :, 1]
    h = jnp.maximum(boxes[:, :, 3], 1e-6)
    m = person_mask.reshape(BT, N).astype(jnp.float32)
    cxr, cyr, mr = cx[:, :, None], cy[:, :, None], m[:, :, None]
    cxc, cyc, hc, mc = cx[:, None, :], cy[:, None, :], h[:, None, :], m[:, None, :]

    row = pl.BlockSpec((G, N, 1), lambda g: (g, 0, 0))
    col = pl.BlockSpec((G, 1, N), lambda g: (g, 0, 0))
    xsp = pl.BlockSpec((G, N, D), lambda g: (g, 0, 0))

    def full(arr):
        return pl.BlockSpec(arr.shape, lambda g: (0,) * arr.ndim)

    w1ea, w1eb, wce = W1e[:D], W1e[D:2 * D], W1e[2 * D:]
    w1na, w1nb = W1n[:D], W1n[D:]
    wts = (w1ea, w1eb, wce, b1e.reshape(1, H), W2e, b2e.reshape(1, D),
           w1na, w1nb, b1n.reshape(1, H), W2n, b2n.reshape(1, D),
           gamma.reshape(1, D), beta.reshape(1, D))

    out = pl.pallas_call(
        _graph_body,
        grid=(BT // G,),
        in_specs=[row, row, row, col, col, col, col, xsp] + [full(w) for w in wts],
        out_specs=xsp,
        out_shape=jax.ShapeDtypeStruct((BT, N, D), jnp.float32),
        interpret=interpret,
    )(cxr, cyr, mr, cxc, cyc, hc, mc, x, *wts)
    return out.reshape(B, T, N, D)
